# baseline (speedup 1.0000x reference)
import functools

import jax
import jax.numpy as jnp
from jax import lax
from jax.experimental import pallas as pl
from jax.experimental.pallas import tpu as pltpu
from jax.experimental.pallas import tpu_sc as plsc

L = 16
NC = 2
NS = 16
NW = NC * NS


def _pe_add_kernel(N, D, rows_per_w, R):
    n_chunks = rows_per_w // R
    mesh = plsc.VectorSubcoreMesh(core_axis_name="c", subcore_axis_name="s")

    @functools.partial(
        pl.kernel,
        out_type=jax.ShapeDtypeStruct((N, D), jnp.float32),
        mesh=mesh,
        scratch_types=[
            pltpu.VMEM((rows_per_w,), jnp.int32),
            pltpu.VMEM((rows_per_w + L,), jnp.float32),
            pltpu.VMEM((R, D), jnp.float32),
            pltpu.VMEM((R, D), jnp.float32),
            pltpu.SemaphoreType.DMA,
            pltpu.SemaphoreType.DMA,
        ],
    )
    def k(x_hbm, pos_hbm, mask_hbm, pe_hbm, out_hbm,
          idx_v, mask_v, x_buf, pe_buf, sem_x, sem_pe):
        wid = lax.axis_index("s") * NC + lax.axis_index("c")
        base = wid * rows_per_w
        pltpu.sync_copy(pos_hbm.at[pl.ds(base, rows_per_w)], idx_v)
        pltpu.sync_copy(mask_hbm.at[pl.ds(base, rows_per_w)],
                        mask_v.at[pl.ds(0, rows_per_w)])
        for c in range(n_chunks):
            r0 = c * R
            cp_x = pltpu.async_copy(x_hbm.at[pl.ds(base + r0, R)], x_buf, sem_x)
            cp_pe = pltpu.async_copy(pe_hbm.at[idx_v.at[pl.ds(r0, R)]],
                                     pe_buf, sem_pe)
            cp_x.wait()
            cp_pe.wait()

            def row_body(r, carry):
                m = mask_v[pl.ds(r0 + r, L)][0]
                for j in range(D // L):
                    sl = pl.ds(j * L, L)
                    x_buf[r, sl] = x_buf[r, sl] + pe_buf[r, sl] * m
                return carry

            lax.fori_loop(0, R, row_body, 0)
            pltpu.sync_copy(x_buf, out_hbm.at[pl.ds(base + r0, R)])

    return k


def kernel(x, positions, token_mask, pe):
    B, S, D = x.shape
    N = B * S
    xf = x.reshape(N, D)
    posf = positions.reshape(N)
    maskf = token_mask.reshape(N).astype(jnp.float32)
    rows_per_w = N // NW
    out = _pe_add_kernel(N, D, rows_per_w, R=32)(xf, posf, maskf, pe)
    return out.reshape(B, S, D)


# speedup vs baseline: 1.3149x; 1.3149x over previous
import functools

import jax
import jax.numpy as jnp
from jax import lax
from jax.experimental import pallas as pl
from jax.experimental.pallas import tpu as pltpu
from jax.experimental.pallas import tpu_sc as plsc

L = 16
NC = 2
NS = 16
NW = NC * NS


def _pe_add_kernel(N, D, rows_per_w, R):
    n_chunks = rows_per_w // R
    mesh = plsc.VectorSubcoreMesh(core_axis_name="c", subcore_axis_name="s")

    @functools.partial(
        pl.kernel,
        out_type=jax.ShapeDtypeStruct((N, D), jnp.float32),
        mesh=mesh,
        scratch_types=[
            pltpu.VMEM((rows_per_w,), jnp.int32),
            pltpu.VMEM((rows_per_w + L,), jnp.float32),
            pltpu.VMEM((R, D), jnp.float32),
            pltpu.VMEM((R, D), jnp.float32),
            pltpu.VMEM((R, D), jnp.float32),
            pltpu.VMEM((R, D), jnp.float32),
            pltpu.SemaphoreType.DMA,
            pltpu.SemaphoreType.DMA,
            pltpu.SemaphoreType.DMA,
            pltpu.SemaphoreType.DMA,
            pltpu.SemaphoreType.DMA,
            pltpu.SemaphoreType.DMA,
        ],
    )
    def k(x_hbm, pos_hbm, mask_hbm, pe_hbm, out_hbm,
          idx_v, mask_v, xb0, xb1, pb0, pb1,
          sem_x0, sem_x1, sem_p0, sem_p1, sem_o0, sem_o1):
        wid = lax.axis_index("s") * NC + lax.axis_index("c")
        base = wid * rows_per_w
        pltpu.sync_copy(pos_hbm.at[pl.ds(base, rows_per_w)], idx_v)
        pltpu.sync_copy(mask_hbm.at[pl.ds(base, rows_per_w)],
                        mask_v.at[pl.ds(0, rows_per_w)])

        def min_body(i, acc):
            return jnp.minimum(acc, mask_v[pl.ds(i * L, L)])

        acc = lax.fori_loop(0, rows_per_w // L, min_body,
                            jnp.full((L,), 1.0, jnp.float32))
        m_min = acc[0]
        for i in range(1, L):
            m_min = jnp.minimum(m_min, acc[i])

        xbufs = (xb0, xb1)
        pbufs = (pb0, pb1)
        sems_x = (sem_x0, sem_x1)
        sems_p = (sem_p0, sem_p1)
        sems_o = (sem_o0, sem_o1)

        def start_loads(c, slot):
            cx = pltpu.async_copy(
                x_hbm.at[pl.ds(base + c * R, R)], xbufs[slot], sems_x[slot])
            cp = pltpu.async_copy(
                pe_hbm.at[idx_v.at[pl.ds(c * R, R)]], pbufs[slot],
                sems_p[slot])
            return cx, cp

        @pl.when(m_min > 0.5)
        def _fast():
            cp_x = [None, None]
            cp_p = [None, None]
            cp_o = [None, None]
            cp_x[0], cp_p[0] = start_loads(0, 0)
            for c in range(n_chunks):
                cur = c & 1
                nxt = 1 - cur
                if c + 1 < n_chunks:
                    if cp_o[nxt] is not None:
                        cp_o[nxt].wait()
                    cp_x[nxt], cp_p[nxt] = start_loads(c + 1, nxt)
                cp_x[cur].wait()
                cp_p[cur].wait()
                xb = xbufs[cur]
                pb = pbufs[cur]

                def row_body(r, carry):
                    for j in range(D // L):
                        sl = pl.ds(j * L, L)
                        plsc.addupdate(xb.at[r, sl], pb[r, sl])
                    return carry

                lax.fori_loop(0, R, row_body, 0)
                cp_o[cur] = pltpu.async_copy(
                    xb, out_hbm.at[pl.ds(base + c * R, R)], sems_o[cur])
            cp_o[(n_chunks - 1) & 1].wait()
            if n_chunks > 1:
                cp_o[n_chunks & 1].wait()

        @pl.when(m_min <= 0.5)
        def _slow():
            for c in range(n_chunks):
                r0 = c * R
                cp_x = pltpu.async_copy(
                    x_hbm.at[pl.ds(base + r0, R)], xb0, sem_x0)
                cp_pe = pltpu.async_copy(
                    pe_hbm.at[idx_v.at[pl.ds(r0, R)]], pb0, sem_p0)
                cp_x.wait()
                cp_pe.wait()

                def row_body(r, carry):
                    m = mask_v[pl.ds(r0 + r, L)][0]
                    for j in range(D // L):
                        sl = pl.ds(j * L, L)
                        plsc.addupdate(xb0.at[r, sl], pb0[r, sl] * m)
                    return carry

                lax.fori_loop(0, R, row_body, 0)
                pltpu.sync_copy(xb0, out_hbm.at[pl.ds(base + r0, R)])

    return k


def kernel(x, positions, token_mask, pe):
    B, S, D = x.shape
    N = B * S
    xf = x.reshape(N, D)
    posf = positions.reshape(N)
    maskf = token_mask.reshape(N).astype(jnp.float32)
    rows_per_w = N // NW
    out = _pe_add_kernel(N, D, rows_per_w, R=16)(xf, posf, maskf, pe)
    return out.reshape(B, S, D)


# speedup vs baseline: 1.4640x; 1.1134x over previous
import functools

import jax
import jax.numpy as jnp
from jax import lax
from jax.experimental import pallas as pl
from jax.experimental.pallas import tpu as pltpu
from jax.experimental.pallas import tpu_sc as plsc

L = 16
NC = 2
NS = 16
NW = NC * NS
NBUF = 3


def _pe_add_kernel(N, D, rows_per_w, R):
    n_chunks = rows_per_w // R
    mesh = plsc.VectorSubcoreMesh(core_axis_name="c", subcore_axis_name="s")

    buf_types = [pltpu.VMEM((R, D), jnp.float32) for _ in range(2 * NBUF)]
    sem_types = [pltpu.SemaphoreType.DMA for _ in range(3 * NBUF)]

    @functools.partial(
        pl.kernel,
        out_type=jax.ShapeDtypeStruct((N, D), jnp.float32),
        mesh=mesh,
        scratch_types=[
            pltpu.VMEM((rows_per_w,), jnp.int32),
            pltpu.VMEM((rows_per_w + L,), jnp.float32),
        ] + buf_types + sem_types,
    )
    def k(x_hbm, pos_hbm, mask_hbm, pe_hbm, out_hbm, idx_v, mask_v, *rest):
        xbufs = rest[0:NBUF]
        pbufs = rest[NBUF:2 * NBUF]
        sems_x = rest[2 * NBUF:2 * NBUF + NBUF]
        sems_p = rest[3 * NBUF:3 * NBUF + NBUF]
        sems_o = rest[4 * NBUF:4 * NBUF + NBUF]

        wid = lax.axis_index("s") * NC + lax.axis_index("c")
        base = wid * rows_per_w
        pltpu.sync_copy(pos_hbm.at[pl.ds(base, rows_per_w)], idx_v)
        pltpu.sync_copy(mask_hbm.at[pl.ds(base, rows_per_w)],
                        mask_v.at[pl.ds(0, rows_per_w)])

        def min_body(i, acc):
            return jnp.minimum(acc, mask_v[pl.ds(i * L, L)])

        acc = lax.fori_loop(0, rows_per_w // L, min_body,
                            jnp.full((L,), 1.0, jnp.float32))
        m_min = acc[0]
        for i in range(1, L):
            m_min = jnp.minimum(m_min, acc[i])

        def start_loads(c, slot):
            cx = pltpu.async_copy(
                x_hbm.at[pl.ds(base + c * R, R)], xbufs[slot], sems_x[slot])
            cp = pltpu.async_copy(
                pe_hbm.at[idx_v.at[pl.ds(c * R, R)]], pbufs[slot],
                sems_p[slot])
            return cx, cp

        @pl.when(m_min > 0.5)
        def _fast():
            cp_x = [None] * NBUF
            cp_p = [None] * NBUF
            cp_o = [None] * NBUF
            for c in range(min(2, n_chunks)):
                cp_x[c], cp_p[c] = start_loads(c, c)
            for c in range(n_chunks):
                s = c % NBUF
                if c + 2 < n_chunks:
                    s2 = (c + 2) % NBUF
                    if cp_o[s2] is not None:
                        cp_o[s2].wait()
                    cp_x[s2], cp_p[s2] = start_loads(c + 2, s2)
                cp_x[s].wait()
                cp_p[s].wait()
                xb = xbufs[s]
                pb = pbufs[s]

                @functools.partial(plsc.parallel_loop, 0, R, unroll=2)
                def combine(r):
                    for j in range(D // L):
                        sl = pl.ds(j * L, L)
                        plsc.addupdate(xb.at[r, sl], pb[r, sl])

                cp_o[s] = pltpu.async_copy(
                    xb, out_hbm.at[pl.ds(base + c * R, R)], sems_o[s])
            for s in range(NBUF):
                if cp_o[s] is not None:
                    cp_o[s].wait()

        @pl.when(m_min <= 0.5)
        def _slow():
            for c in range(n_chunks):
                r0 = c * R
                cp_x = pltpu.async_copy(
                    x_hbm.at[pl.ds(base + r0, R)], xbufs[0], sems_x[0])
                cp_pe = pltpu.async_copy(
                    pe_hbm.at[idx_v.at[pl.ds(r0, R)]], pbufs[0], sems_p[0])
                cp_x.wait()
                cp_pe.wait()

                def row_body(r, carry):
                    m = mask_v[pl.ds(r0 + r, L)][0]
                    for j in range(D // L):
                        sl = pl.ds(j * L, L)
                        plsc.addupdate(xbufs[0].at[r, sl], pbufs[0][r, sl] * m)
                    return carry

                lax.fori_loop(0, R, row_body, 0)
                pltpu.sync_copy(xbufs[0], out_hbm.at[pl.ds(base + r0, R)])

    return k


def kernel(x, positions, token_mask, pe):
    B, S, D = x.shape
    N = B * S
    xf = x.reshape(N, D)
    posf = positions.reshape(N)
    maskf = token_mask.reshape(N).astype(jnp.float32)
    rows_per_w = N // NW
    out = _pe_add_kernel(N, D, rows_per_w, R=16)(xf, posf, maskf, pe)
    return out.reshape(B, S, D)
